# baseline (speedup 1.0000x reference)
import functools

import jax
import jax.numpy as jnp
from jax import lax
from jax.experimental import pallas as pl
from jax.experimental.pallas import tpu as pltpu
from jax.experimental.pallas import tpu_sc as plsc

N = 10000
E = 320000
D = 128

NC = 2
NS = 16
NW = NC * NS
EPT = E // NW
K = 80
KA = 48
KB = K - KA
CHUNKS = EPT // K
ZCH = 80
NZCH = N // ZCH
ZITER = (NZCH + NS - 1) // NS

_mesh = plsc.VectorSubcoreMesh(core_axis_name="c", subcore_axis_name="s")

_GDN = lax.GatherDimensionNumbers(
    offset_dims=(), collapsed_slice_dims=(0,), start_index_map=(0,))


@functools.partial(
    pl.kernel,
    mesh=_mesh,
    out_type=jax.ShapeDtypeStruct((NC, N, D), jnp.float32),
    scratch_types=[
        pltpu.VMEM_SHARED((N, D), jnp.float32),
        pltpu.VMEM((EPT,), jnp.int32),
        pltpu.VMEM((K,), jnp.int32),
        pltpu.VMEM((K,), jnp.int32),
        pltpu.VMEM((K,), jnp.int32),
        pltpu.VMEM((K,), jnp.float32),
        pltpu.VMEM((K,), jnp.float32),
        pltpu.VMEM((K,), jnp.float32),
        pltpu.VMEM((K, D), jnp.float32),
        pltpu.VMEM((K, D), jnp.float32),
        pltpu.VMEM((K, D), jnp.float32),
        pltpu.SemaphoreType.DMA,
        pltpu.SemaphoreType.DMA,
        pltpu.SemaphoreType.DMA,
        pltpu.SemaphoreType.DMA,
        pltpu.SemaphoreType.DMA,
        pltpu.SemaphoreType.DMA,
        pltpu.SemaphoreType.DMA,
        pltpu.SemaphoreType.DMA,
        pltpu.SemaphoreType.DMA,
    ],
)
def _sc_segsum(x_hbm, edges_hbm, ew_hbm, out_hbm,
               acc, src_all, dst0, dst1, dst2,
               ew0, ew1, ew2, rows0, rows1, rows2,
               gsem0, gsem1, gsem2, dsem0, dsem1, dsem2,
               ssem0, ssem1, ssem2):
    c = lax.axis_index("c")
    s = lax.axis_index("s")
    wid = c * NS + s
    ebase = wid * EPT

    dst_b = (dst0, dst1, dst2)
    ew_b = (ew0, ew1, ew2)
    rows_b = (rows0, rows1, rows2)
    gsem_b = (gsem0, gsem1, gsem2)
    dsem_b = (dsem0, dsem1, dsem2)
    ssem_b = (ssem0, ssem1, ssem2)

    pltpu.sync_copy(edges_hbm.at[pl.ds(E + ebase, EPT)], src_all)

    def issue(j, b):
        eb = ebase + j * K
        pltpu.async_copy(edges_hbm.at[pl.ds(eb, K)], dst_b[b], dsem_b[b])
        pltpu.async_copy(ew_hbm.at[pl.ds(eb, K)], ew_b[b], dsem_b[b])
        pltpu.async_copy(x_hbm.at[src_all.at[pl.ds(j * K, K)]], rows_b[b],
                         gsem_b[b])

    def wait_scatter(b):
        pltpu.make_async_copy(x_hbm.at[pl.ds(0, K)], rows_b[b],
                              ssem_b[b]).wait()

    def do_chunk(j, b, first=False):
        pltpu.make_async_copy(edges_hbm.at[pl.ds(0, K)], dst_b[b],
                              dsem_b[b]).wait()
        pltpu.make_async_copy(ew_hbm.at[pl.ds(0, K)], ew_b[b],
                              dsem_b[b]).wait()
        pltpu.make_async_copy(x_hbm.at[pl.ds(0, K)], rows_b[b],
                              gsem_b[b]).wait()

        rows = rows_b[b]
        ew = ew_b[b]

        for g in range(K // 16):
            w16 = ew[pl.ds(g * 16, 16)]
            for l in range(16):
                w = lax.gather(w16, jnp.full((16, 1), l, jnp.int32),
                               dimension_numbers=_GDN, slice_sizes=(1,),
                               mode=lax.GatherScatterMode.PROMISE_IN_BOUNDS)
                row = rows.at[g * 16 + l]
                for c8 in range(D // 16):
                    sl = pl.ds(c8 * 16, 16)
                    row[sl] = row[sl] * w
        pltpu.async_copy(rows, acc.at[dst_b[b]], ssem_b[b], add=True)

        nxt = j + 2
        bn = (b + 2) % 3

        @pl.when(nxt < CHUNKS)
        def _():
            if not first:
                wait_scatter(bn)
            issue(nxt, bn)

    issue(0, 0)
    issue(1, 1)

    zeros16 = jnp.zeros((16,), jnp.float32)

    def zrow(r, carry):
        for c8 in range(D // 16):
            rows2[r, pl.ds(c8 * 16, 16)] = zeros16
        return carry

    lax.fori_loop(0, ZCH, zrow, 0)

    def zcopy(i, carry):
        z = s + i * NS

        @pl.when(z < NZCH)
        def _():
            pltpu.async_copy(rows2, acc.at[pl.ds(z * ZCH, ZCH)], ssem0)

        return carry

    lax.fori_loop(0, ZITER, zcopy, 0)

    def zdrain(i, carry):
        @pl.when(s + i * NS < NZCH)
        def _():
            pltpu.make_async_copy(x_hbm.at[pl.ds(0, ZCH)], rows2,
                                  ssem0).wait()

        return carry

    lax.fori_loop(0, ZITER, zdrain, 0)
    plsc.subcore_barrier()

    do_chunk(0, 0, first=True)

    def triple(t, carry):
        j = 3 * t + 1
        do_chunk(j, 1)
        do_chunk(j + 1, 2)
        do_chunk(j + 2, 0)
        return carry

    lax.fori_loop(0, (CHUNKS - 2) // 3, triple, 0)
    do_chunk(CHUNKS - 1, 1)
    wait_scatter(0)
    wait_scatter(1)
    wait_scatter(2)
    plsc.subcore_barrier()

    def ocopy(i, carry):
        z = s + i * NS

        @pl.when(z < NZCH)
        def _():
            base = z * ZCH
            pltpu.async_copy(acc.at[pl.ds(base, ZCH)],
                             out_hbm.at[c, pl.ds(base, ZCH)], ssem0)

        return carry

    lax.fori_loop(0, ZITER, ocopy, 0)

    def odrain(i, carry):
        @pl.when(s + i * NS < NZCH)
        def _():
            pltpu.make_async_copy(acc.at[pl.ds(0, ZCH)],
                                  out_hbm.at[c, pl.ds(0, ZCH)],
                                  ssem0).wait()

        return carry

    lax.fori_loop(0, ZITER, odrain, 0)


BLK = 1000


def _tc_self_body(x_ref, ws_ref, bs_ref, bn_ref, o_ref):
    dn = (((1,), (1,)), ((), ()))
    o_ref[...] = (lax.dot_general(x_ref[...], ws_ref[...], dn,
                                  preferred_element_type=jnp.float32)
                  + bs_ref[...] + bn_ref[...])


def _tc_final_body(s_ref, p0_ref, p1_ref, wn_ref, o_ref):
    neigh = p0_ref[...] + p1_ref[...]
    dn = (((1,), (1,)), ((), ()))
    out = s_ref[...] + lax.dot_general(neigh, wn_ref[...], dn,
                                       preferred_element_type=jnp.float32)
    o_ref[...] = jnp.maximum(out, 0.0)


_ROWSPEC = pl.BlockSpec((BLK, D), lambda i: (i, 0))
_FULLSPEC = pl.BlockSpec((D, D), lambda i: (0, 0))
_BIASSPEC = pl.BlockSpec((1, D), lambda i: (0, 0))


@jax.jit
def _tc_self(x, W_self, b_self, b_neigh):
    return pl.pallas_call(
        _tc_self_body,
        grid=(N // BLK,),
        in_specs=[_ROWSPEC, _FULLSPEC, _BIASSPEC, _BIASSPEC],
        out_specs=_ROWSPEC,
        out_shape=jax.ShapeDtypeStruct((N, D), jnp.float32),
    )(x, W_self, b_self.reshape(1, D), b_neigh.reshape(1, D))


@jax.jit
def _tc_final(self_part, p0, p1, W_neigh):
    return pl.pallas_call(
        _tc_final_body,
        grid=(N // BLK,),
        in_specs=[_ROWSPEC, _ROWSPEC, _ROWSPEC, _FULLSPEC],
        out_specs=_ROWSPEC,
        out_shape=jax.ShapeDtypeStruct((N, D), jnp.float32),
    )(self_part, p0, p1, W_neigh)


def kernel(x, edge_index, edge_weight, W_self, b_self, W_neigh, b_neigh):
    edges_flat = edge_index.astype(jnp.int32).reshape(-1)
    partials = _sc_segsum(x, edges_flat, edge_weight)
    self_part = _tc_self(x, W_self, b_self, b_neigh)
    return _tc_final(self_part, partials[0], partials[1], W_neigh)


# speedup vs baseline: 1.2947x; 1.2947x over previous
import functools

import jax
import jax.numpy as jnp
from jax import lax
from jax.experimental import pallas as pl
from jax.experimental.pallas import tpu as pltpu
from jax.experimental.pallas import tpu_sc as plsc

N = 10000
E = 320000
D = 128

NC = 2
NS = 16
NW = NC * NS
EPT = E // NW
K = 80
KA = 48
KB = K - KA
CHUNKS = EPT // K
ZCH = 80
NZCH = N // ZCH
ZITER = (NZCH + NS - 1) // NS

_mesh = plsc.VectorSubcoreMesh(core_axis_name="c", subcore_axis_name="s")

_GDN = lax.GatherDimensionNumbers(
    offset_dims=(), collapsed_slice_dims=(0,), start_index_map=(0,))


@functools.partial(
    pl.kernel,
    mesh=_mesh,
    out_type=jax.ShapeDtypeStruct((NC, N, D), jnp.float32),
    scratch_types=[
        pltpu.VMEM_SHARED((N, D), jnp.float32),
        pltpu.VMEM((EPT,), jnp.int32),
        pltpu.VMEM((K,), jnp.int32),
        pltpu.VMEM((K,), jnp.int32),
        pltpu.VMEM((K,), jnp.int32),
        pltpu.VMEM((K,), jnp.float32),
        pltpu.VMEM((K,), jnp.float32),
        pltpu.VMEM((K,), jnp.float32),
        pltpu.VMEM((K, D), jnp.float32),
        pltpu.VMEM((K, D), jnp.float32),
        pltpu.VMEM((K, D), jnp.float32),
        pltpu.SemaphoreType.DMA,
        pltpu.SemaphoreType.DMA,
        pltpu.SemaphoreType.DMA,
        pltpu.SemaphoreType.DMA,
        pltpu.SemaphoreType.DMA,
        pltpu.SemaphoreType.DMA,
        pltpu.SemaphoreType.DMA,
        pltpu.SemaphoreType.DMA,
        pltpu.SemaphoreType.DMA,
    ],
)
def _sc_segsum(x_hbm, edges_hbm, ew_hbm, out_hbm,
               acc, src_all, dst0, dst1, dst2,
               ew0, ew1, ew2, rows0, rows1, rows2,
               gsem0, gsem1, gsem2, dsem0, dsem1, dsem2,
               ssem0, ssem1, ssem2):
    c = lax.axis_index("c")
    s = lax.axis_index("s")
    wid = c * NS + s
    ebase = wid * EPT

    dst_b = (dst0, dst1, dst2)
    ew_b = (ew0, ew1, ew2)
    rows_b = (rows0, rows1, rows2)
    gsem_b = (gsem0, gsem1, gsem2)
    dsem_b = (dsem0, dsem1, dsem2)
    ssem_b = (ssem0, ssem1, ssem2)

    pltpu.sync_copy(edges_hbm.at[pl.ds(E + ebase, EPT)], src_all)

    def issue(j, b):
        eb = ebase + j * K
        pltpu.async_copy(edges_hbm.at[pl.ds(eb, K)], dst_b[b], dsem_b[b])
        pltpu.async_copy(ew_hbm.at[pl.ds(eb, K)], ew_b[b], dsem_b[b])
        pltpu.async_copy(x_hbm.at[src_all.at[pl.ds(j * K, K)]], rows_b[b],
                         gsem_b[b])

    def wait_scatter(b):
        pltpu.make_async_copy(x_hbm.at[pl.ds(0, K)], rows_b[b],
                              ssem_b[b]).wait()

    def do_chunk(j, b, first=False):
        pltpu.make_async_copy(edges_hbm.at[pl.ds(0, K)], dst_b[b],
                              dsem_b[b]).wait()
        pltpu.make_async_copy(ew_hbm.at[pl.ds(0, K)], ew_b[b],
                              dsem_b[b]).wait()
        pltpu.make_async_copy(x_hbm.at[pl.ds(0, K)], rows_b[b],
                              gsem_b[b]).wait()

        rows = rows_b[b]
        ew = ew_b[b]

        def egroup(g, icarry):
            w16 = ew[pl.ds(g * 16, 16)]
            for l in range(16):
                w = lax.gather(w16, jnp.full((16, 1), l, jnp.int32),
                               dimension_numbers=_GDN, slice_sizes=(1,),
                               mode=lax.GatherScatterMode.PROMISE_IN_BOUNDS)
                row = rows.at[g * 16 + l]
                for c8 in range(D // 16):
                    sl = pl.ds(c8 * 16, 16)
                    row[sl] = row[sl] * w
            return icarry

        lax.fori_loop(0, K // 16, egroup, 0)
        pltpu.async_copy(rows, acc.at[dst_b[b]], ssem_b[b], add=True)

        nxt = j + 2
        bn = (b + 2) % 3

        @pl.when(nxt < CHUNKS)
        def _():
            if not first:
                wait_scatter(bn)
            issue(nxt, bn)

    issue(0, 0)
    issue(1, 1)

    zeros16 = jnp.zeros((16,), jnp.float32)

    def zrow(r, carry):
        for c8 in range(D // 16):
            rows2[r, pl.ds(c8 * 16, 16)] = zeros16
        return carry

    lax.fori_loop(0, ZCH, zrow, 0)

    def zcopy(i, carry):
        z = s + i * NS

        @pl.when(z < NZCH)
        def _():
            pltpu.async_copy(rows2, acc.at[pl.ds(z * ZCH, ZCH)], ssem0)

        return carry

    lax.fori_loop(0, ZITER, zcopy, 0)

    def zdrain(i, carry):
        @pl.when(s + i * NS < NZCH)
        def _():
            pltpu.make_async_copy(x_hbm.at[pl.ds(0, ZCH)], rows2,
                                  ssem0).wait()

        return carry

    lax.fori_loop(0, ZITER, zdrain, 0)
    plsc.subcore_barrier()

    do_chunk(0, 0, first=True)

    def triple(t, carry):
        j = 3 * t + 1
        do_chunk(j, 1)
        do_chunk(j + 1, 2)
        do_chunk(j + 2, 0)
        return carry

    lax.fori_loop(0, (CHUNKS - 2) // 3, triple, 0)
    do_chunk(CHUNKS - 1, 1)
    wait_scatter(0)
    wait_scatter(1)
    wait_scatter(2)
    plsc.subcore_barrier()

    def ocopy(i, carry):
        z = s + i * NS

        @pl.when(z < NZCH)
        def _():
            base = z * ZCH
            pltpu.async_copy(acc.at[pl.ds(base, ZCH)],
                             out_hbm.at[c, pl.ds(base, ZCH)], ssem0)

        return carry

    lax.fori_loop(0, ZITER, ocopy, 0)

    def odrain(i, carry):
        @pl.when(s + i * NS < NZCH)
        def _():
            pltpu.make_async_copy(acc.at[pl.ds(0, ZCH)],
                                  out_hbm.at[c, pl.ds(0, ZCH)],
                                  ssem0).wait()

        return carry

    lax.fori_loop(0, ZITER, odrain, 0)


BLK = 1000


def _tc_self_body(x_ref, ws_ref, bs_ref, bn_ref, o_ref):
    dn = (((1,), (1,)), ((), ()))
    o_ref[...] = (lax.dot_general(x_ref[...], ws_ref[...], dn,
                                  preferred_element_type=jnp.float32)
                  + bs_ref[...] + bn_ref[...])


def _tc_final_body(s_ref, p0_ref, p1_ref, wn_ref, o_ref):
    neigh = p0_ref[...] + p1_ref[...]
    dn = (((1,), (1,)), ((), ()))
    out = s_ref[...] + lax.dot_general(neigh, wn_ref[...], dn,
                                       preferred_element_type=jnp.float32)
    o_ref[...] = jnp.maximum(out, 0.0)


_ROWSPEC = pl.BlockSpec((BLK, D), lambda i: (i, 0))
_FULLSPEC = pl.BlockSpec((D, D), lambda i: (0, 0))
_BIASSPEC = pl.BlockSpec((1, D), lambda i: (0, 0))


@jax.jit
def _tc_self(x, W_self, b_self, b_neigh):
    return pl.pallas_call(
        _tc_self_body,
        grid=(N // BLK,),
        in_specs=[_ROWSPEC, _FULLSPEC, _BIASSPEC, _BIASSPEC],
        out_specs=_ROWSPEC,
        out_shape=jax.ShapeDtypeStruct((N, D), jnp.float32),
    )(x, W_self, b_self.reshape(1, D), b_neigh.reshape(1, D))


@jax.jit
def _tc_final(self_part, p0, p1, W_neigh):
    return pl.pallas_call(
        _tc_final_body,
        grid=(N // BLK,),
        in_specs=[_ROWSPEC, _ROWSPEC, _ROWSPEC, _FULLSPEC],
        out_specs=_ROWSPEC,
        out_shape=jax.ShapeDtypeStruct((N, D), jnp.float32),
    )(self_part, p0, p1, W_neigh)


def kernel(x, edge_index, edge_weight, W_self, b_self, W_neigh, b_neigh):
    edges_flat = edge_index.astype(jnp.int32).reshape(-1)
    partials = _sc_segsum(x, edges_flat, edge_weight)
    self_part = _tc_self(x, W_self, b_self, b_neigh)
    return _tc_final(self_part, partials[0], partials[1], W_neigh)


# speedup vs baseline: 1.3138x; 1.0148x over previous
import functools

import jax
import jax.numpy as jnp
from jax import lax
from jax.experimental import pallas as pl
from jax.experimental.pallas import tpu as pltpu
from jax.experimental.pallas import tpu_sc as plsc

N = 10000
E = 320000
D = 128

NC = 2
NS = 16
NW = NC * NS
EPT = E // NW
K = 80
KA = 48
KB = K - KA
CHUNKS = EPT // K
ZCH = 80
NZCH = N // ZCH
ZITER = (NZCH + NS - 1) // NS

_mesh = plsc.VectorSubcoreMesh(core_axis_name="c", subcore_axis_name="s")

_GDN = lax.GatherDimensionNumbers(
    offset_dims=(), collapsed_slice_dims=(0,), start_index_map=(0,))


@functools.partial(
    pl.kernel,
    mesh=_mesh,
    out_type=jax.ShapeDtypeStruct((NC, N, D), jnp.float32),
    scratch_types=[
        pltpu.VMEM_SHARED((N, D), jnp.float32),
        pltpu.VMEM((EPT,), jnp.int32),
        pltpu.VMEM((K,), jnp.int32),
        pltpu.VMEM((K,), jnp.int32),
        pltpu.VMEM((K,), jnp.int32),
        pltpu.VMEM((K,), jnp.float32),
        pltpu.VMEM((K,), jnp.float32),
        pltpu.VMEM((K,), jnp.float32),
        pltpu.VMEM((K, D), jnp.float32),
        pltpu.VMEM((K, D), jnp.float32),
        pltpu.VMEM((K, D), jnp.float32),
        pltpu.SemaphoreType.DMA,
        pltpu.SemaphoreType.DMA,
        pltpu.SemaphoreType.DMA,
        pltpu.SemaphoreType.DMA,
        pltpu.SemaphoreType.DMA,
        pltpu.SemaphoreType.DMA,
        pltpu.SemaphoreType.DMA,
        pltpu.SemaphoreType.DMA,
        pltpu.SemaphoreType.DMA,
    ],
)
def _sc_segsum(x_hbm, edges_hbm, ew_hbm, out_hbm,
               acc, src_all, dst0, dst1, dst2,
               ew0, ew1, ew2, rows0, rows1, rows2,
               gsem0, gsem1, gsem2, dsem0, dsem1, dsem2,
               ssem0, ssem1, ssem2):
    c = lax.axis_index("c")
    s = lax.axis_index("s")
    wid = c * NS + s
    ebase = wid * EPT

    dst_b = (dst0, dst1, dst2)
    ew_b = (ew0, ew1, ew2)
    rows_b = (rows0, rows1, rows2)
    gsem_b = (gsem0, gsem1, gsem2)
    dsem_b = (dsem0, dsem1, dsem2)
    ssem_b = (ssem0, ssem1, ssem2)

    pltpu.sync_copy(edges_hbm.at[pl.ds(E + ebase, EPT)], src_all)

    def issue(j, b):
        eb = ebase + j * K
        pltpu.async_copy(x_hbm.at[src_all.at[pl.ds(j * K, K)]], rows_b[b],
                         gsem_b[b])
        pltpu.async_copy(edges_hbm.at[pl.ds(eb, K)], dst_b[b], dsem_b[b])
        pltpu.async_copy(ew_hbm.at[pl.ds(eb, K)], ew_b[b], dsem_b[b])

    def wait_scatter(b):
        pltpu.make_async_copy(x_hbm.at[pl.ds(0, K)], rows_b[b],
                              ssem_b[b]).wait()

    def do_chunk(j, b, first=False):
        pltpu.make_async_copy(edges_hbm.at[pl.ds(0, K)], dst_b[b],
                              dsem_b[b]).wait()
        pltpu.make_async_copy(ew_hbm.at[pl.ds(0, K)], ew_b[b],
                              dsem_b[b]).wait()
        pltpu.make_async_copy(x_hbm.at[pl.ds(0, K)], rows_b[b],
                              gsem_b[b]).wait()

        rows = rows_b[b]
        ew = ew_b[b]

        def egroup(g, icarry):
            w16 = ew[pl.ds(g * 16, 16)]
            for l in range(16):
                w = lax.gather(w16, jnp.full((16, 1), l, jnp.int32),
                               dimension_numbers=_GDN, slice_sizes=(1,),
                               mode=lax.GatherScatterMode.PROMISE_IN_BOUNDS)
                row = rows.at[g * 16 + l]
                for c8 in range(D // 16):
                    sl = pl.ds(c8 * 16, 16)
                    row[sl] = row[sl] * w
            return icarry

        lax.fori_loop(0, K // 16, egroup, 0)
        pltpu.async_copy(rows, acc.at[dst_b[b]], ssem_b[b], add=True)

        nxt = j + 2
        bn = (b + 2) % 3

        @pl.when(nxt < CHUNKS)
        def _():
            if not first:
                wait_scatter(bn)
            issue(nxt, bn)

    issue(0, 0)
    issue(1, 1)

    zeros16 = jnp.zeros((16,), jnp.float32)

    def zrow(r, carry):
        for c8 in range(D // 16):
            rows2[r, pl.ds(c8 * 16, 16)] = zeros16
        return carry

    lax.fori_loop(0, ZCH, zrow, 0)

    def zcopy(i, carry):
        z = s + i * NS

        @pl.when(z < NZCH)
        def _():
            pltpu.async_copy(rows2, acc.at[pl.ds(z * ZCH, ZCH)], ssem0)

        return carry

    lax.fori_loop(0, ZITER, zcopy, 0)

    def zdrain(i, carry):
        @pl.when(s + i * NS < NZCH)
        def _():
            pltpu.make_async_copy(x_hbm.at[pl.ds(0, ZCH)], rows2,
                                  ssem0).wait()

        return carry

    lax.fori_loop(0, ZITER, zdrain, 0)
    plsc.subcore_barrier()

    do_chunk(0, 0, first=True)

    def triple(t, carry):
        j = 3 * t + 1
        do_chunk(j, 1)
        do_chunk(j + 1, 2)
        do_chunk(j + 2, 0)
        return carry

    lax.fori_loop(0, (CHUNKS - 2) // 3, triple, 0)
    do_chunk(CHUNKS - 1, 1)
    wait_scatter(0)
    wait_scatter(1)
    wait_scatter(2)
    plsc.subcore_barrier()

    def ocopy(i, carry):
        z = s + i * NS

        @pl.when(z < NZCH)
        def _():
            base = z * ZCH
            pltpu.async_copy(acc.at[pl.ds(base, ZCH)],
                             out_hbm.at[c, pl.ds(base, ZCH)], ssem0)

        return carry

    lax.fori_loop(0, ZITER, ocopy, 0)

    def odrain(i, carry):
        @pl.when(s + i * NS < NZCH)
        def _():
            pltpu.make_async_copy(acc.at[pl.ds(0, ZCH)],
                                  out_hbm.at[c, pl.ds(0, ZCH)],
                                  ssem0).wait()

        return carry

    lax.fori_loop(0, ZITER, odrain, 0)


BLK = 2000


def _tc_self_body(x_ref, ws_ref, bs_ref, bn_ref, o_ref):
    dn = (((1,), (1,)), ((), ()))
    o_ref[...] = (lax.dot_general(x_ref[...], ws_ref[...], dn,
                                  preferred_element_type=jnp.float32)
                  + bs_ref[...] + bn_ref[...])


def _tc_final_body(s_ref, p0_ref, p1_ref, wn_ref, o_ref):
    neigh = p0_ref[...] + p1_ref[...]
    dn = (((1,), (1,)), ((), ()))
    out = s_ref[...] + lax.dot_general(neigh, wn_ref[...], dn,
                                       preferred_element_type=jnp.float32)
    o_ref[...] = jnp.maximum(out, 0.0)


_ROWSPEC = pl.BlockSpec((BLK, D), lambda i: (i, 0))
_FULLSPEC = pl.BlockSpec((D, D), lambda i: (0, 0))
_BIASSPEC = pl.BlockSpec((1, D), lambda i: (0, 0))


@jax.jit
def _tc_self(x, W_self, b_self, b_neigh):
    return pl.pallas_call(
        _tc_self_body,
        grid=(N // BLK,),
        in_specs=[_ROWSPEC, _FULLSPEC, _BIASSPEC, _BIASSPEC],
        out_specs=_ROWSPEC,
        out_shape=jax.ShapeDtypeStruct((N, D), jnp.float32),
    )(x, W_self, b_self.reshape(1, D), b_neigh.reshape(1, D))


@jax.jit
def _tc_final(self_part, p0, p1, W_neigh):
    return pl.pallas_call(
        _tc_final_body,
        grid=(N // BLK,),
        in_specs=[_ROWSPEC, _ROWSPEC, _ROWSPEC, _FULLSPEC],
        out_specs=_ROWSPEC,
        out_shape=jax.ShapeDtypeStruct((N, D), jnp.float32),
    )(self_part, p0, p1, W_neigh)


def kernel(x, edge_index, edge_weight, W_self, b_self, W_neigh, b_neigh):
    edges_flat = edge_index.astype(jnp.int32).reshape(-1)
    partials = _sc_segsum(x, edges_flat, edge_weight)
    self_part = _tc_self(x, W_self, b_self, b_neigh)
    return _tc_final(self_part, partials[0], partials[1], W_neigh)


# speedup vs baseline: 1.3274x; 1.0103x over previous
import functools

import jax
import jax.numpy as jnp
from jax import lax
from jax.experimental import pallas as pl
from jax.experimental.pallas import tpu as pltpu
from jax.experimental.pallas import tpu_sc as plsc

N = 10000
E = 320000
D = 128

NC = 2
NS = 16
NW = NC * NS
EPT = E // NW
K = 80
KA = 48
KB = K - KA
CHUNKS = EPT // K
ZCH = 80
NZCH = N // ZCH
ZITER = (NZCH + NS - 1) // NS

_mesh = plsc.VectorSubcoreMesh(core_axis_name="c", subcore_axis_name="s")

_GDN = lax.GatherDimensionNumbers(
    offset_dims=(), collapsed_slice_dims=(0,), start_index_map=(0,))


@functools.partial(
    pl.kernel,
    mesh=_mesh,
    out_type=jax.ShapeDtypeStruct((NC, N, D), jnp.float32),
    scratch_types=[
        pltpu.VMEM_SHARED((N, D), jnp.float32),
        pltpu.VMEM((EPT,), jnp.int32),
        pltpu.VMEM((K,), jnp.int32),
        pltpu.VMEM((K,), jnp.int32),
        pltpu.VMEM((K,), jnp.int32),
        pltpu.VMEM((K,), jnp.float32),
        pltpu.VMEM((K,), jnp.float32),
        pltpu.VMEM((K,), jnp.float32),
        pltpu.VMEM((K, D), jnp.float32),
        pltpu.VMEM((K, D), jnp.float32),
        pltpu.VMEM((K, D), jnp.float32),
        pltpu.SemaphoreType.DMA,
        pltpu.SemaphoreType.DMA,
        pltpu.SemaphoreType.DMA,
        pltpu.SemaphoreType.DMA,
        pltpu.SemaphoreType.DMA,
        pltpu.SemaphoreType.DMA,
        pltpu.SemaphoreType.DMA,
        pltpu.SemaphoreType.DMA,
        pltpu.SemaphoreType.DMA,
    ],
)
def _sc_segsum(x_hbm, edges_hbm, ew_hbm, out_hbm,
               acc, src_all, dst0, dst1, dst2,
               ew0, ew1, ew2, rows0, rows1, rows2,
               gsem0, gsem1, gsem2, dsem0, dsem1, dsem2,
               ssem0, ssem1, ssem2):
    c = lax.axis_index("c")
    s = lax.axis_index("s")
    wid = c * NS + s
    ebase = wid * EPT

    dst_b = (dst0, dst1, dst2)
    ew_b = (ew0, ew1, ew2)
    rows_b = (rows0, rows1, rows2)
    gsem_b = (gsem0, gsem1, gsem2)
    dsem_b = (dsem0, dsem1, dsem2)
    ssem_b = (ssem0, ssem1, ssem2)

    pltpu.sync_copy(edges_hbm.at[pl.ds(E + ebase, EPT)], src_all)

    def issue(j, b):
        eb = ebase + j * K
        pltpu.async_copy(x_hbm.at[src_all.at[pl.ds(j * K, K)]], rows_b[b],
                         gsem_b[b])
        pltpu.async_copy(edges_hbm.at[pl.ds(eb, K)], dst_b[b], dsem_b[b])
        pltpu.async_copy(ew_hbm.at[pl.ds(eb, K)], ew_b[b], dsem_b[b])

    def wait_scatter(b):
        pltpu.make_async_copy(x_hbm.at[pl.ds(0, K)], rows_b[b],
                              ssem_b[b]).wait()

    def do_chunk(j, b, first=False):
        pltpu.make_async_copy(edges_hbm.at[pl.ds(0, K)], dst_b[b],
                              dsem_b[b]).wait()
        pltpu.make_async_copy(ew_hbm.at[pl.ds(0, K)], ew_b[b],
                              dsem_b[b]).wait()
        pltpu.make_async_copy(x_hbm.at[pl.ds(0, K)], rows_b[b],
                              gsem_b[b]).wait()

        rows = rows_b[b]
        ew = ew_b[b]

        def egroup(g, icarry):
            w16 = ew[pl.ds(g * 16, 16)]
            for l in range(16):
                w = lax.gather(w16, jnp.full((16, 1), l, jnp.int32),
                               dimension_numbers=_GDN, slice_sizes=(1,),
                               mode=lax.GatherScatterMode.PROMISE_IN_BOUNDS)
                row = rows.at[g * 16 + l]
                for c8 in range(D // 16):
                    sl = pl.ds(c8 * 16, 16)
                    row[sl] = row[sl] * w
            return icarry

        lax.fori_loop(0, K // 16, egroup, 0)
        pltpu.async_copy(rows, acc.at[dst_b[b]], ssem_b[b], add=True)

        nxt = j + 2
        bn = (b + 2) % 3

        @pl.when(nxt < CHUNKS)
        def _():
            if not first:
                wait_scatter(bn)
            issue(nxt, bn)

    issue(0, 0)
    issue(1, 1)

    zeros16 = jnp.zeros((16,), jnp.float32)

    def zrow(r, carry):
        for c8 in range(D // 16):
            rows2[r, pl.ds(c8 * 16, 16)] = zeros16
        return carry

    lax.fori_loop(0, ZCH, zrow, 0)

    def zcopy(i, carry):
        z = s + i * NS

        @pl.when(z < NZCH)
        def _():
            pltpu.async_copy(rows2, acc.at[pl.ds(z * ZCH, ZCH)], ssem0)

        return carry

    lax.fori_loop(0, ZITER, zcopy, 0)

    def zdrain(i, carry):
        @pl.when(s + i * NS < NZCH)
        def _():
            pltpu.make_async_copy(x_hbm.at[pl.ds(0, ZCH)], rows2,
                                  ssem0).wait()

        return carry

    lax.fori_loop(0, ZITER, zdrain, 0)
    plsc.subcore_barrier()

    do_chunk(0, 0, first=True)

    def triple(t, carry):
        j = 3 * t + 1
        do_chunk(j, 1)
        do_chunk(j + 1, 2)
        do_chunk(j + 2, 0)
        return carry

    lax.fori_loop(0, (CHUNKS - 2) // 3, triple, 0)
    do_chunk(CHUNKS - 1, 1)
    wait_scatter(0)
    wait_scatter(1)
    wait_scatter(2)
    plsc.subcore_barrier()

    def ocopy(i, carry):
        z = s + i * NS

        @pl.when(z < NZCH)
        def _():
            base = z * ZCH
            pltpu.async_copy(acc.at[pl.ds(base, ZCH)],
                             out_hbm.at[c, pl.ds(base, ZCH)], ssem0)

        return carry

    lax.fori_loop(0, ZITER, ocopy, 0)

    def odrain(i, carry):
        @pl.when(s + i * NS < NZCH)
        def _():
            pltpu.make_async_copy(acc.at[pl.ds(0, ZCH)],
                                  out_hbm.at[c, pl.ds(0, ZCH)],
                                  ssem0).wait()

        return carry

    lax.fori_loop(0, ZITER, odrain, 0)


BLK = 5000


def _tc_self_body(x_ref, ws_ref, bs_ref, bn_ref, o_ref):
    dn = (((1,), (1,)), ((), ()))
    o_ref[...] = (lax.dot_general(x_ref[...], ws_ref[...], dn,
                                  preferred_element_type=jnp.float32)
                  + bs_ref[...] + bn_ref[...])


def _tc_final_body(s_ref, p0_ref, p1_ref, wn_ref, o_ref):
    neigh = p0_ref[...] + p1_ref[...]
    dn = (((1,), (1,)), ((), ()))
    out = s_ref[...] + lax.dot_general(neigh, wn_ref[...], dn,
                                       preferred_element_type=jnp.float32)
    o_ref[...] = jnp.maximum(out, 0.0)


_ROWSPEC = pl.BlockSpec((BLK, D), lambda i: (i, 0))
_FULLSPEC = pl.BlockSpec((D, D), lambda i: (0, 0))
_BIASSPEC = pl.BlockSpec((1, D), lambda i: (0, 0))


@jax.jit
def _tc_self(x, W_self, b_self, b_neigh):
    return pl.pallas_call(
        _tc_self_body,
        grid=(N // BLK,),
        in_specs=[_ROWSPEC, _FULLSPEC, _BIASSPEC, _BIASSPEC],
        out_specs=_ROWSPEC,
        out_shape=jax.ShapeDtypeStruct((N, D), jnp.float32),
    )(x, W_self, b_self.reshape(1, D), b_neigh.reshape(1, D))


@jax.jit
def _tc_final(self_part, p0, p1, W_neigh):
    return pl.pallas_call(
        _tc_final_body,
        grid=(N // BLK,),
        in_specs=[_ROWSPEC, _ROWSPEC, _ROWSPEC, _FULLSPEC],
        out_specs=_ROWSPEC,
        out_shape=jax.ShapeDtypeStruct((N, D), jnp.float32),
    )(self_part, p0, p1, W_neigh)


def kernel(x, edge_index, edge_weight, W_self, b_self, W_neigh, b_neigh):
    edges_flat = edge_index.astype(jnp.int32).reshape(-1)
    partials = _sc_segsum(x, edges_flat, edge_weight)
    self_part = _tc_self(x, W_self, b_self, b_neigh)
    return _tc_final(self_part, partials[0], partials[1], W_neigh)
